# Initial kernel scaffold; baseline (speedup 1.0000x reference)
import jax
import jax.numpy as jnp
from jax.experimental import pallas as pl


def kernel(edge_features, graph_embedding, selected_commodity, candidate_paths, path_mask, W1, b1, W2, b2):
    raise NotImplementedError("write your pallas kernel here")


# speedup vs baseline: 1.2006x; 1.2006x over previous
import functools

import jax
import jax.numpy as jnp
from jax import lax
from jax.experimental import pallas as pl
from jax.experimental.pallas import tpu as pltpu
from jax.experimental.pallas import tpu_sc as plsc

LANES = 16


def _sc_gather_pool_body(n_dim, c_dim, n_edges, rows_per_w, paths_per_w,
                         table_hbm, u_hbm, v_hbm, base_hbm, out_hbm,
                         u_v, v_v, idx_v, base_v, rows_v, acc_v, sem):
    wid = lax.axis_index("s") * 2 + lax.axis_index("c")
    row_base = wid * rows_per_w

    pltpu.sync_copy(u_hbm.at[pl.ds(row_base, rows_per_w)], u_v)
    pltpu.sync_copy(v_hbm.at[pl.ds(row_base, rows_per_w)], v_v)
    pltpu.sync_copy(base_hbm.at[wid], base_v)

    base_vec = base_v[...]

    n_chunks = rows_per_w // LANES
    for m in range(n_chunks):
        sl = pl.ds(m * LANES, LANES)
        uu = u_v[sl]
        vv = v_v[sl]
        idx_v[sl] = (uu * n_dim + vv) * c_dim + base_vec

    half = rows_per_w // 2
    cp0 = pltpu.async_copy(table_hbm.at[idx_v.at[pl.ds(0, half)]],
                           rows_v.at[pl.ds(0, half)], sem)
    cp1 = pltpu.async_copy(table_hbm.at[idx_v.at[pl.ds(half, half)]],
                           rows_v.at[pl.ds(half, half)], sem)
    cp0.wait()
    cp1.wait()

    inv = 1.0 / float(n_edges)

    def pool_one(j, carry):
        r0 = j * n_edges
        for k in range(256 // LANES):
            cs = pl.ds(k * LANES, LANES)
            acc = rows_v[r0, cs]
            for e in range(1, n_edges):
                acc = acc + rows_v[r0 + e, cs]
            acc_v[j, cs] = acc * inv
        return carry

    lax.fori_loop(0, paths_per_w, pool_one, 0)

    pltpu.sync_copy(acc_v, out_hbm.at[pl.ds(wid * paths_per_w, paths_per_w)])


def _sc_gather_pool(table, u, v, base_rows, n_edges, n_dim, c_dim):
    total_rows = u.shape[0]
    n_paths = total_rows // n_edges
    h_dim = table.shape[1]
    nw = 32
    rows_per_w = total_rows // nw
    paths_per_w = n_paths // nw

    mesh = plsc.VectorSubcoreMesh(core_axis_name="c", subcore_axis_name="s")
    body = functools.partial(_sc_gather_pool_body, n_dim, c_dim, n_edges,
                             rows_per_w, paths_per_w)
    f = pl.kernel(
        body,
        mesh=mesh,
        out_type=jax.ShapeDtypeStruct((n_paths, h_dim), jnp.float32),
        scratch_types=[
            pltpu.VMEM((rows_per_w,), jnp.int32),
            pltpu.VMEM((rows_per_w,), jnp.int32),
            pltpu.VMEM((rows_per_w,), jnp.int32),
            pltpu.VMEM((LANES,), jnp.int32),
            pltpu.VMEM((rows_per_w, h_dim), jnp.float32),
            pltpu.VMEM((paths_per_w, h_dim), jnp.float32),
            pltpu.SemaphoreType.DMA,
        ],
    )
    return f(table, u, v, base_rows)


def _tc_mlp_body(b_dim, p_dim, pf_ref, g_ref, w1_ref, b1_ref, w2_ref, b2_ref,
                 mask_ref, probs_ref, logp_ref, ent_ref):
    h_dim = g_ref.shape[1]
    pf = pf_ref[...]
    w1a = w1_ref[0:h_dim, :]
    w1b = w1_ref[h_dim:2 * h_dim, :]
    h1 = jnp.dot(pf, w1a, preferred_element_type=jnp.float32)
    hg = jnp.dot(g_ref[...], w1b, preferred_element_type=jnp.float32)
    h = h1.reshape(b_dim, p_dim, -1) + hg[:, None, :] + b1_ref[...][None, None, :]
    h = jnp.maximum(h, 0.0)
    s = jnp.dot(h.reshape(b_dim * p_dim, -1), w2_ref[...],
                preferred_element_type=jnp.float32)
    s = s.reshape(b_dim, p_dim) + b2_ref[...]
    m = mask_ref[...] > 0.0
    s = jnp.where(m, s, -jnp.inf)
    mx = jnp.max(s, axis=1, keepdims=True)
    e = jnp.exp(s - mx)
    denom = jnp.sum(e, axis=1, keepdims=True)
    probs = e / denom
    logp = s - mx - jnp.log(denom)
    probs_ref[...] = probs
    logp_ref[...] = logp
    ent_ref[...] = -jnp.sum(probs * jnp.where(m, logp, 0.0), axis=1)


def _tc_mlp(path_feat, g, w1, b1, w2, b2, mask_f):
    b_dim, p_dim = mask_f.shape
    body = functools.partial(_tc_mlp_body, b_dim, p_dim)
    return pl.pallas_call(
        body,
        out_shape=[
            jax.ShapeDtypeStruct((b_dim, p_dim), jnp.float32),
            jax.ShapeDtypeStruct((b_dim, p_dim), jnp.float32),
            jax.ShapeDtypeStruct((b_dim,), jnp.float32),
        ],
    )(path_feat, g, w1, b1, w2, b2, mask_f)


def kernel(edge_features, graph_embedding, selected_commodity, candidate_paths,
           path_mask, W1, b1, W2, b2):
    B, N, _, C, H = edge_features.shape
    P, L = candidate_paths.shape[1], candidate_paths.shape[2]
    n_edges = L - 1

    table = edge_features.reshape(B * N * N * C, H)
    u = candidate_paths[:, :, :-1].reshape(-1).astype(jnp.int32)
    v = candidate_paths[:, :, 1:].reshape(-1).astype(jnp.int32)
    base = (jnp.arange(32, dtype=jnp.int32) // 2) * (N * N * C) \
        + selected_commodity.astype(jnp.int32)[jnp.arange(32) // 2]
    base_rows = jnp.broadcast_to(base[:, None], (32, LANES))

    path_feat = _sc_gather_pool(table, u, v, base_rows, n_edges, N, C)
    probs, logp, ent = _tc_mlp(path_feat, graph_embedding, W1, b1, W2, b2,
                               path_mask.astype(jnp.float32))
    return probs, logp, ent


# speedup vs baseline: 1.3224x; 1.1014x over previous
import functools

import jax
import jax.numpy as jnp
from jax import lax
from jax.experimental import pallas as pl
from jax.experimental.pallas import tpu as pltpu
from jax.experimental.pallas import tpu_sc as plsc

LANES = 16


def _sc_gather_pool_body(n_dim, c_dim, n_edges, l_dim, paths_per_w, h_dim,
                         table_hbm, paths_hbm, base_hbm, out_hbm,
                         nodes_v, idx_v, base_v, rows_v, acc_v, sem, sem2):
    wid = lax.axis_index("s") * 2 + lax.axis_index("c")
    nodes_per_w = paths_per_w * l_dim
    rows_per_w = nodes_per_w

    cp_n = pltpu.async_copy(paths_hbm.at[pl.ds(wid * nodes_per_w, nodes_per_w)],
                            nodes_v.at[pl.ds(0, nodes_per_w)], sem2)
    cp_b = pltpu.async_copy(base_hbm.at[wid], base_v, sem2)
    cp_n.wait()
    cp_b.wait()
    nodes_v[pl.ds(nodes_per_w, LANES)] = jnp.zeros((LANES,), jnp.int32)
    base_vec = base_v[...]

    n_chunks = nodes_per_w // LANES

    def idx_body(t, carry):
        uu = nodes_v[pl.ds(t * LANES, LANES)]
        vv = nodes_v[pl.ds(t * LANES + 1, LANES)]
        idx_v[pl.ds(t * LANES, LANES)] = (uu * n_dim + vv) * c_dim + base_vec
        return carry

    lax.fori_loop(0, n_chunks, idx_body, 0)

    half = rows_per_w // 2
    cp0 = pltpu.async_copy(table_hbm.at[idx_v.at[pl.ds(0, half)]],
                           rows_v.at[pl.ds(0, half)], sem)
    cp1 = pltpu.async_copy(table_hbm.at[idx_v.at[pl.ds(half, half)]],
                           rows_v.at[pl.ds(half, half)], sem)

    inv = 1.0 / float(n_edges)
    col_chunks = h_dim // LANES

    def pool_one(j, carry):
        r0 = j * l_dim

        def pool_col(k, c2):
            cs = pl.ds(k * LANES, LANES)
            acc = rows_v[r0, cs]
            for e in range(1, n_edges):
                acc = acc + rows_v[r0 + e, cs]
            acc_v[j, cs] = acc * inv
            return c2

        lax.fori_loop(0, col_chunks, pool_col, 0)
        return carry

    cp0.wait()
    lax.fori_loop(0, paths_per_w // 2, pool_one, 0)
    cp1.wait()
    lax.fori_loop(paths_per_w // 2, paths_per_w, pool_one, 0)

    pltpu.sync_copy(acc_v, out_hbm.at[pl.ds(wid * paths_per_w, paths_per_w)])


def _sc_gather_pool(table, paths_flat, base_rows, n_edges, n_dim, c_dim, l_dim):
    n_paths = paths_flat.shape[0] // l_dim
    h_dim = table.shape[1]
    nw = 32
    paths_per_w = n_paths // nw
    rows_per_w = paths_per_w * l_dim

    mesh = plsc.VectorSubcoreMesh(core_axis_name="c", subcore_axis_name="s")
    body = functools.partial(_sc_gather_pool_body, n_dim, c_dim, n_edges,
                             l_dim, paths_per_w, h_dim)
    f = pl.kernel(
        body,
        mesh=mesh,
        out_type=jax.ShapeDtypeStruct((n_paths, h_dim), jnp.float32),
        scratch_types=[
            pltpu.VMEM((rows_per_w + LANES,), jnp.int32),
            pltpu.VMEM((rows_per_w,), jnp.int32),
            pltpu.VMEM((LANES,), jnp.int32),
            pltpu.VMEM((rows_per_w, h_dim), jnp.float32),
            pltpu.VMEM((paths_per_w, h_dim), jnp.float32),
            pltpu.SemaphoreType.DMA,
            pltpu.SemaphoreType.DMA,
        ],
    )
    return f(table, paths_flat, base_rows)


def _tc_mlp_body(b_dim, p_dim, pf_ref, g_ref, w1_ref, b1_ref, w2_ref, b2_ref,
                 mask_ref, probs_ref, logp_ref, ent_ref):
    h_dim = g_ref.shape[1]
    pf = pf_ref[...]
    w1a = w1_ref[0:h_dim, :]
    w1b = w1_ref[h_dim:2 * h_dim, :]
    h1 = jnp.dot(pf, w1a, preferred_element_type=jnp.float32)
    hg = jnp.dot(g_ref[...], w1b, preferred_element_type=jnp.float32)
    h = h1.reshape(b_dim, p_dim, -1) + hg[:, None, :] + b1_ref[...][None, None, :]
    h = jnp.maximum(h, 0.0)
    s = jnp.dot(h.reshape(b_dim * p_dim, -1), w2_ref[...],
                preferred_element_type=jnp.float32)
    s = s.reshape(b_dim, p_dim) + b2_ref[...]
    m = mask_ref[...] > 0.0
    s = jnp.where(m, s, -jnp.inf)
    mx = jnp.max(s, axis=1, keepdims=True)
    e = jnp.exp(s - mx)
    denom = jnp.sum(e, axis=1, keepdims=True)
    probs = e / denom
    logp = s - mx - jnp.log(denom)
    probs_ref[...] = probs
    logp_ref[...] = logp
    ent_ref[...] = -jnp.sum(probs * jnp.where(m, logp, 0.0), axis=1)


def _tc_mlp(path_feat, g, w1, b1, w2, b2, mask_f):
    b_dim, p_dim = mask_f.shape
    body = functools.partial(_tc_mlp_body, b_dim, p_dim)
    return pl.pallas_call(
        body,
        out_shape=[
            jax.ShapeDtypeStruct((b_dim, p_dim), jnp.float32),
            jax.ShapeDtypeStruct((b_dim, p_dim), jnp.float32),
            jax.ShapeDtypeStruct((b_dim,), jnp.float32),
        ],
    )(path_feat, g, w1, b1, w2, b2, mask_f)


def kernel(edge_features, graph_embedding, selected_commodity, candidate_paths,
           path_mask, W1, b1, W2, b2):
    B, N, _, C, H = edge_features.shape
    P, L = candidate_paths.shape[1], candidate_paths.shape[2]
    n_edges = L - 1

    table = edge_features.reshape(B * N * N * C, H)
    paths_flat = candidate_paths.reshape(-1).astype(jnp.int32)
    base = (jnp.arange(32, dtype=jnp.int32) // 2) * (N * N * C) \
        + selected_commodity.astype(jnp.int32)[jnp.arange(32) // 2]
    base_rows = jnp.broadcast_to(base[:, None], (32, LANES))

    path_feat = _sc_gather_pool(table, paths_flat, base_rows, n_edges, N, C, L)
    probs, logp, ent = _tc_mlp(path_feat, graph_embedding, W1, b1, W2, b2,
                               path_mask.astype(jnp.float32))
    return probs, logp, ent
